# baseline (speedup 1.0000x reference)
import functools

import jax
import jax.numpy as jnp
from jax import lax
from jax.experimental import pallas as pl
from jax.experimental.pallas import tpu as pltpu
from jax.experimental.pallas import tpu_sc as plsc

N_F = 26
FH = 13
VOCAB_SZ = 100000
VPAD = 102400
D = 16
B = 16384

NC = 2
NS = 16
NW = NC * NS
LANES = 16

BPW = B // NW
XPW = BPW * N_F
LPW = BPW * FH
VPW = LPW // LANES
GRP = BPW // LANES


def _tc_reduce_body(t_ref, s_ref):
    s_ref[...] = jnp.sum(t_ref[0, :, :], axis=0)


def _make_sc_body(h, with_partial):
    def body(*refs):
        if with_partial:
            (s_hbm, xflat_hbm, alpha_hbm, qmap_hbm, omap_hbm, part_hbm,
             out_hbm, xbuf, qbuf, obuf, idxbuf, gbuf, alo, pbuf, outb,
             sem) = refs
        else:
            (s_hbm, xflat_hbm, alpha_hbm, qmap_hbm, omap_hbm,
             out_hbm, xbuf, qbuf, obuf, idxbuf, gbuf, alo, pbuf, outb,
             sem) = refs
        wid = lax.axis_index("s") * NC + lax.axis_index("c")

        pltpu.sync_copy(xflat_hbm.at[pl.ds(wid * XPW, XPW)], xbuf)
        pltpu.sync_copy(qmap_hbm, qbuf)
        pltpu.sync_copy(omap_hbm, obuf)
        if with_partial:
            pltpu.sync_copy(part_hbm.at[pl.ds(wid * BPW, BPW)], pbuf)

        pltpu.sync_copy(alpha_hbm, alo)
        for j in range(2):
            a = alo[pl.ds(j * LANES, LANES)]
            e = jnp.exp(a + a)
            alo[pl.ds(j * LANES, LANES)] = (e - 1.0) / (e + 1.0)
        ta = [plsc.load_gather(alo, [jnp.full((LANES,), h * FH + f, jnp.int32)])
              for f in range(FH)]

        def idx_body(k, carry):
            s = k * LANES
            xv = plsc.load_gather(xbuf, [qbuf[pl.ds(s, LANES)]])
            idxbuf[pl.ds(s, LANES)] = xv + obuf[pl.ds(s, LANES)]
            return carry
        lax.fori_loop(0, VPW, idx_body, 0)

        pltpu.async_copy(s_hbm.at[idxbuf], gbuf, sem).wait()

        iov = lax.iota(jnp.int32, LANES)
        io13 = iov * FH

        def group_body(g, carry):
            gb = io13 + g * (LANES * FH)
            acc = plsc.load_gather(gbuf, [gb]) * ta[0]
            for f in range(1, FH):
                acc = acc + plsc.load_gather(gbuf, [gb + f]) * ta[f]
            if with_partial:
                acc = acc + pbuf[pl.ds(g * LANES, LANES)]
            outb[pl.ds(g * LANES, LANES)] = acc
            return carry
        lax.fori_loop(0, GRP, group_body, 0)

        pltpu.sync_copy(outb, out_hbm.at[pl.ds(wid * BPW, BPW)])
    return body


@jax.jit
def kernel(X, tables, alpha):
    tt = jnp.transpose(tables, (0, 2, 1))
    s_halves = [
        pl.pallas_call(
            _tc_reduce_body,
            grid=(FH,),
            in_specs=[pl.BlockSpec((1, D, VPAD),
                                   lambda f, h=h: (h * FH + f, 0, 0))],
            out_specs=pl.BlockSpec((VPAD,), lambda f: (f,)),
            out_shape=jax.ShapeDtypeStruct((FH * VPAD,), jnp.float32),
        )(tt)
        for h in range(2)
    ]

    xflat = X.reshape(B * N_F)
    alpha_pad = jnp.pad(alpha, (0, 2 * LANES - N_F))
    t = jnp.arange(LPW, dtype=jnp.int32)
    qmap = (t // FH) * N_F + (t % FH)
    omap = (t % FH) * VPAD

    mesh = plsc.VectorSubcoreMesh(core_axis_name="c", subcore_axis_name="s")
    cp = pltpu.CompilerParams(
        needs_layout_passes=False, use_tc_tiling_on_sc=False)
    scratch = [
        pltpu.VMEM((XPW,), jnp.int32),
        pltpu.VMEM((LPW,), jnp.int32),
        pltpu.VMEM((LPW,), jnp.int32),
        pltpu.VMEM((LPW,), jnp.int32),
        pltpu.VMEM((LPW,), jnp.float32),
        pltpu.VMEM((2 * LANES,), jnp.float32),
        pltpu.VMEM((BPW,), jnp.float32),
        pltpu.VMEM((BPW,), jnp.float32),
        pltpu.SemaphoreType.DMA,
    ]
    part = pl.kernel(
        _make_sc_body(0, False),
        out_type=jax.ShapeDtypeStruct((B,), jnp.float32),
        mesh=mesh, compiler_params=cp, scratch_types=scratch,
    )(s_halves[0], xflat, alpha_pad, qmap, omap)
    out = pl.kernel(
        _make_sc_body(1, True),
        out_type=jax.ShapeDtypeStruct((B,), jnp.float32),
        mesh=mesh, compiler_params=cp, scratch_types=scratch,
    )(s_halves[1], xflat, alpha_pad, qmap + FH, omap, part)
    return out[:, None]


# speedup vs baseline: 1.1389x; 1.1389x over previous
import functools

import jax
import jax.numpy as jnp
from jax import lax
from jax.experimental import pallas as pl
from jax.experimental.pallas import tpu as pltpu
from jax.experimental.pallas import tpu_sc as plsc

N_F = 26
FH = 13
VOCAB_SZ = 100000
VPAD = 102400
D = 16
B = 16384

NC = 2
NS = 16
NW = NC * NS
LANES = 16

BPW = B // NW
LPW = BPW * FH
VPB = BPW // LANES
GRP = BPW // LANES


def _tc_reduce_body(t_ref, s_ref):
    s_ref[...] = jnp.sum(t_ref[0, :, :], axis=0)


def _make_sc_body(h, with_partial):
    def body(*refs):
        if with_partial:
            (s_hbm, xt_hbm, alpha_hbm, part_hbm,
             out_hbm, xbuf, idxbuf, gbuf, alo, pbuf, outb, sem) = refs
        else:
            (s_hbm, xt_hbm, alpha_hbm,
             out_hbm, xbuf, idxbuf, gbuf, alo, pbuf, outb, sem) = refs
        wid = lax.axis_index("s") * NC + lax.axis_index("c")

        for fp in range(FH):
            pltpu.sync_copy(xt_hbm.at[h * FH + fp, pl.ds(wid * BPW, BPW)],
                            xbuf.at[pl.ds(fp * BPW, BPW)])
        if with_partial:
            pltpu.sync_copy(part_hbm.at[pl.ds(wid * BPW, BPW)], pbuf)

        pltpu.sync_copy(alpha_hbm, alo)
        for j in range(2):
            a = alo[pl.ds(j * LANES, LANES)]
            e = jnp.exp(a + a)
            alo[pl.ds(j * LANES, LANES)] = (e - 1.0) / (e + 1.0)
        ta = [plsc.load_gather(alo, [jnp.full((LANES,), h * FH + f, jnp.int32)])
              for f in range(FH)]

        def idx_body(k, carry):
            s = k * LANES
            fof = (k // (BPW // LANES)) * VPAD
            idxbuf[pl.ds(s, LANES)] = xbuf[pl.ds(s, LANES)] + fof
            return carry
        lax.fori_loop(0, FH * VPB, idx_body, 0)

        pltpu.async_copy(s_hbm.at[idxbuf], gbuf, sem).wait()

        def group_body(g, carry):
            o = g * LANES
            acc = gbuf[pl.ds(o, LANES)] * ta[0]
            for f in range(1, FH):
                acc = acc + gbuf[pl.ds(f * BPW + o, LANES)] * ta[f]
            if with_partial:
                acc = acc + pbuf[pl.ds(o, LANES)]
            outb[pl.ds(o, LANES)] = acc
            return carry
        lax.fori_loop(0, GRP, group_body, 0)

        pltpu.sync_copy(outb, out_hbm.at[pl.ds(wid * BPW, BPW)])
    return body


@jax.jit
def kernel(X, tables, alpha):
    tt = jnp.transpose(tables, (0, 2, 1))
    xt = jnp.transpose(X, (1, 0))
    s_halves = [
        pl.pallas_call(
            _tc_reduce_body,
            grid=(FH,),
            in_specs=[pl.BlockSpec((1, D, VPAD),
                                   lambda f, h=h: (h * FH + f, 0, 0))],
            out_specs=pl.BlockSpec((VPAD,), lambda f: (f,)),
            out_shape=jax.ShapeDtypeStruct((FH * VPAD,), jnp.float32),
        )(tt)
        for h in range(2)
    ]

    alpha_pad = jnp.pad(alpha, (0, 2 * LANES - N_F))

    mesh = plsc.VectorSubcoreMesh(core_axis_name="c", subcore_axis_name="s")
    cp = pltpu.CompilerParams(
        needs_layout_passes=False, use_tc_tiling_on_sc=False)
    scratch = [
        pltpu.VMEM((LPW,), jnp.int32),
        pltpu.VMEM((LPW,), jnp.int32),
        pltpu.VMEM((LPW,), jnp.float32),
        pltpu.VMEM((2 * LANES,), jnp.float32),
        pltpu.VMEM((BPW,), jnp.float32),
        pltpu.VMEM((BPW,), jnp.float32),
        pltpu.SemaphoreType.DMA,
    ]
    part = pl.kernel(
        _make_sc_body(0, False),
        out_type=jax.ShapeDtypeStruct((B,), jnp.float32),
        mesh=mesh, compiler_params=cp, scratch_types=scratch,
    )(s_halves[0], xt, alpha_pad)
    out = pl.kernel(
        _make_sc_body(1, True),
        out_type=jax.ShapeDtypeStruct((B,), jnp.float32),
        mesh=mesh, compiler_params=cp, scratch_types=scratch,
    )(s_halves[1], xt, alpha_pad, part)
    return out[:, None]
